# baseline (speedup 1.0000x reference)
import functools
import jax
import jax.numpy as jnp
from jax import lax
from jax.experimental import pallas as pl
from jax.experimental.pallas import tpu as pltpu
from jax.experimental.pallas import tpu_sc as plsc

IN_FEATURES = 2048
OUT_FEATURES = 2048
R = 8
LORA_ALPHA = 16
E = 8
TOP_K = 2
T = 4096
ER = E * R
SCALING = LORA_ALPHA / R

NW = 32
TPW = T // NW
NG = TPW // 16


NV = TPW * E // 16


def _route_one(l_hbm, w_hbm, p_hbm, l_v, w_v, s_v, base, wid):
    pltpu.sync_copy(l_hbm.at[pl.ds(base * E, TPW * E)], l_v)

    lane = lax.iota(jnp.int32, 16)
    e_id = lane % E
    tok0 = lane - e_id
    one = jnp.ones((16,), jnp.float32)
    zero = jnp.zeros((16,), jnp.float32)
    perms = [tok0 + (e_id + d) % E for d in range(1, E)]
    ties = [e_id >= (E - d) for d in range(1, E)]

    gdn = lax.GatherDimensionNumbers(offset_dims=(), collapsed_slice_dims=(0,),
                                     start_index_map=(0,))

    def take(v, idx):
        return lax.gather(v, idx[:, None], gdn, (1,),
                          mode=lax.GatherScatterMode.PROMISE_IN_BOUNDS)

    def body(g, acc):
        cur = l_v[pl.ds(g * 16, 16)]
        others = [take(cur, idx) for idx in perms]
        m = cur
        for o in others:
            m = jnp.maximum(m, o)
        ex = jnp.exp(cur - m)
        total = ex
        for idx in perms:
            total = total + take(ex, idx)
        p = ex / total
        rank = zero
        for o, tie in zip(others, ties):
            rank = rank + jnp.where(o > cur, one, zero)
            rank = rank + jnp.where((o == cur) & tie, one, zero)
        selp = jnp.where(rank < (TOP_K - 0.5), p, zero)
        denom = selp
        for idx in perms:
            denom = denom + take(selp, idx)
        w_v[pl.ds(g * 16, 16)] = selp / jnp.maximum(denom, 1e-30)
        return acc + p

    acc = lax.fori_loop(0, NV, body, zero)

    s_v[0, :] = acc
    for e in range(1, E):
        s_v[e, :] = zero

    pltpu.sync_copy(w_v, w_hbm.at[pl.ds(base * E, TPW * E)])
    pltpu.sync_copy(s_v, p_hbm.at[pl.ds(wid * E, E)])


def _sc_route_body(la, lb, wa, wb, pA, pB, la_v, wa_v, lb_v, wb_v, sa_v, sb_v):
    wid = lax.axis_index("s") * 2 + lax.axis_index("c")
    base = wid * TPW
    _route_one(la, wa, pA, la_v, wa_v, sa_v, base, wid)
    _route_one(lb, wb, pB, lb_v, wb_v, sb_v, base, wid)


_SC_ROUTE_CACHE = []


def _sc_route(laT, lbT):
    if not _SC_ROUTE_CACHE:
        _SC_ROUTE_CACHE.append(functools.partial(
            pl.kernel,
            mesh=plsc.VectorSubcoreMesh(core_axis_name="c", subcore_axis_name="s"),
            out_type=[
                jax.ShapeDtypeStruct((T * E,), jnp.float32),
                jax.ShapeDtypeStruct((T * E,), jnp.float32),
                jax.ShapeDtypeStruct((NW * E, 16), jnp.float32),
                jax.ShapeDtypeStruct((NW * E, 16), jnp.float32),
            ],
            scratch_types=[
                pltpu.VMEM((TPW * E,), jnp.float32),
                pltpu.VMEM((TPW * E,), jnp.float32),
                pltpu.VMEM((TPW * E,), jnp.float32),
                pltpu.VMEM((TPW * E,), jnp.float32),
                pltpu.VMEM((E, 16), jnp.float32),
                pltpu.VMEM((E, 16), jnp.float32),
            ],
        )(_sc_route_body))
    return _SC_ROUTE_CACHE[0](laT, lbT)


TB = 512
DN = (((1,), (0,)), ((), ()))
DT = (((0,), (0,)), ((), ()))


DR = (((1,), (1,)), ((), ()))


def _tc_a_body(x_ref, WaF_ref, A_ref):
    A_ref[...] = lax.dot_general(x_ref[...], WaF_ref[...], DR,
                                 preferred_element_type=jnp.float32,
                                 precision=jax.lax.Precision.DEFAULT)


def _tc_a(flat_x, WaF, interpret=False):
    return pl.pallas_call(
        _tc_a_body,
        grid=(T // TB,),
        in_specs=[
            pl.BlockSpec((TB, IN_FEATURES), lambda i: (i, 0)),
            pl.BlockSpec((ER, IN_FEATURES), lambda i: (0, 0)),
        ],
        out_specs=pl.BlockSpec((TB, ER), lambda i: (i, 0)),
        out_shape=jax.ShapeDtypeStruct((T, ER), jnp.float32),
        interpret=interpret,
    )(flat_x, WaF)


def _tc_body(A_ref, wa_ref, wb_ref, WbF_ref, pA_ref, pB_ref,
             out_ref, auxA_ref, auxB_ref):
    i = pl.program_id(0)
    f32 = jnp.float32
    hi = jax.lax.Precision.HIGHEST

    rows = lax.broadcasted_iota(jnp.int32, (E, ER), 0)
    cols = lax.broadcasted_iota(jnp.int32, (E, ER), 1)
    E1 = jnp.where(cols // R == rows, 1.0, 0.0).astype(f32)
    c1 = lax.broadcasted_iota(jnp.int32, (ER, ER), 0)
    c2 = lax.broadcasted_iota(jnp.int32, (ER, ER), 1)
    Km = jnp.where(c1 % R == c2 % R, 1.0, 0.0).astype(f32)

    lo = jax.lax.Precision.DEFAULT
    wa_exp = lax.dot_general(wa_ref[...], E1, DN,
                             preferred_element_type=f32, precision=hi)
    mid_tile = lax.dot_general(A_ref[...] * wa_exp, Km, DN,
                               preferred_element_type=f32, precision=hi)
    wb_exp = lax.dot_general(wb_ref[...], E1, DN,
                             preferred_element_type=f32, precision=hi)
    out_ref[...] = lax.dot_general(mid_tile * wb_exp, WbF_ref[...], DN,
                                   preferred_element_type=f32,
                                   precision=lo) * SCALING

    @pl.when(i == pl.num_programs(0) - 1)
    def _aux():
        for p_ref, aux_ref in ((pA_ref, auxA_ref), (pB_ref, auxB_ref)):
            pm16 = jnp.sum(p_ref[...], axis=0, keepdims=True) / T
            pm = pm16[:, :E] + pm16[:, E:]
            mu = jnp.sum(pm) / E
            d = pm - mu
            aux_ref[...] = jnp.broadcast_to(E * jnp.sum(d * d) / (E - 1), (1, 1))


def _tc_combine(A, wa, wb, WbF, pA, pB, interpret=False):
    grid = (T // TB,)
    return pl.pallas_call(
        _tc_body,
        grid=grid,
        in_specs=[
            pl.BlockSpec((TB, ER), lambda i: (i, 0)),
            pl.BlockSpec((TB, E), lambda i: (i, 0)),
            pl.BlockSpec((TB, E), lambda i: (i, 0)),
            pl.BlockSpec((ER, OUT_FEATURES), lambda i: (0, 0)),
            pl.BlockSpec((NW * E, 16), lambda i: (0, 0)),
            pl.BlockSpec((NW * E, 16), lambda i: (0, 0)),
        ],
        out_specs=[
            pl.BlockSpec((TB, OUT_FEATURES), lambda i: (i, 0)),
            pl.BlockSpec((1, 1), lambda i: (0, 0)),
            pl.BlockSpec((1, 1), lambda i: (0, 0)),
        ],
        out_shape=[
            jax.ShapeDtypeStruct((T, OUT_FEATURES), jnp.float32),
            jax.ShapeDtypeStruct((1, 1), jnp.float32),
            jax.ShapeDtypeStruct((1, 1), jnp.float32),
        ],
        interpret=interpret,
    )(A, wa, wb, WbF, pA, pB)


def kernel(x, router_logits_a, router_logits_b, Wa, Wb):
    batch, seq, _ = x.shape
    flat_x = x.reshape(T, IN_FEATURES)
    WaF = Wa.reshape(ER, IN_FEATURES)
    WbF = Wb.transpose(0, 2, 1).reshape(ER, OUT_FEATURES)

    wa_f, wb_f, pA, pB = _sc_route(router_logits_a.reshape(T * E),
                                   router_logits_b.reshape(T * E))
    A = _tc_a(flat_x, WaF)
    out_flat, auxA, auxB = _tc_combine(A, wa_f.reshape(T, E),
                                       wb_f.reshape(T, E), WbF, pA, pB)
    return (out_flat.reshape(batch, seq, OUT_FEATURES), auxA[0, 0], auxB[0, 0])


# speedup vs baseline: 1.0178x; 1.0178x over previous
import functools
import jax
import jax.numpy as jnp
from jax import lax
from jax.experimental import pallas as pl
from jax.experimental.pallas import tpu as pltpu
from jax.experimental.pallas import tpu_sc as plsc

IN_FEATURES = 2048
OUT_FEATURES = 2048
R = 8
LORA_ALPHA = 16
E = 8
TOP_K = 2
T = 4096
ER = E * R
SCALING = LORA_ALPHA / R

NW = 32
TPW = T // NW
NG = TPW // 16


def _route_group(l, acc):
    m = l[0]
    for e in range(1, E):
        m = jnp.maximum(m, l[e])
    ex = [jnp.exp(l[e] - m) for e in range(E)]
    total = ex[0]
    for e in range(1, E):
        total = total + ex[e]
    p = [ex[e] / total for e in range(E)]
    new_acc = [acc[e] + p[e] for e in range(E)]

    one = jnp.ones((16,), jnp.float32)
    zero = jnp.zeros((16,), jnp.float32)
    w = []
    sels = []
    for e in range(E):
        rank = zero
        for j in range(E):
            if j == e:
                continue
            gt = jnp.where(l[j] > l[e], one, zero)
            rank = rank + gt
            if j < e:
                rank = rank + jnp.where(l[j] == l[e], one, zero)
        sels.append(rank < (TOP_K - 0.5))
    denom = zero
    for e in range(E):
        denom = denom + jnp.where(sels[e], p[e], zero)
    for e in range(E):
        w.append(jnp.where(sels[e], p[e] / denom, zero))
    return w, new_acc


def _route_one(l_hbm, w_hbm, p_hbm, l_v, w_v, s_v, base, wid):
    pltpu.sync_copy(l_hbm.at[:, pl.ds(base, TPW)], l_v)

    def body(g, acc):
        l = [l_v[e, pl.ds(g * 16, 16)] for e in range(E)]
        w, acc = _route_group(l, list(acc))
        for e in range(E):
            w_v[e, pl.ds(g * 16, 16)] = w[e]
        return tuple(acc)

    zero = jnp.zeros((16,), jnp.float32)
    acc = lax.fori_loop(0, NG, body, tuple(zero for _ in range(E)))

    for e in range(E):
        s_v[e, :] = acc[e]

    pltpu.sync_copy(w_v, w_hbm.at[:, pl.ds(base, TPW)])
    pltpu.sync_copy(s_v, p_hbm.at[pl.ds(wid * E, E)])


def _sc_route_body(la, lb, wa, wb, pA, pB, la_v, wa_v, lb_v, wb_v, sa_v, sb_v):
    wid = lax.axis_index("s") * 2 + lax.axis_index("c")
    base = wid * TPW
    _route_one(la, wa, pA, la_v, wa_v, sa_v, base, wid)
    _route_one(lb, wb, pB, lb_v, wb_v, sb_v, base, wid)


_SC_ROUTE_CACHE = []


def _sc_route(laT, lbT):
    if not _SC_ROUTE_CACHE:
        _SC_ROUTE_CACHE.append(functools.partial(
            pl.kernel,
            mesh=plsc.VectorSubcoreMesh(core_axis_name="c", subcore_axis_name="s"),
            out_type=[
                jax.ShapeDtypeStruct((E, T), jnp.float32),
                jax.ShapeDtypeStruct((E, T), jnp.float32),
                jax.ShapeDtypeStruct((NW * E, 16), jnp.float32),
                jax.ShapeDtypeStruct((NW * E, 16), jnp.float32),
            ],
            scratch_types=[
                pltpu.VMEM((E, TPW), jnp.float32),
                pltpu.VMEM((E, TPW), jnp.float32),
                pltpu.VMEM((E, TPW), jnp.float32),
                pltpu.VMEM((E, TPW), jnp.float32),
                pltpu.VMEM((E, 16), jnp.float32),
                pltpu.VMEM((E, 16), jnp.float32),
            ],
        )(_sc_route_body))
    return _SC_ROUTE_CACHE[0](laT, lbT)


TB = 512
DN = (((1,), (0,)), ((), ()))
DT = (((0,), (0,)), ((), ()))


DR = (((1,), (1,)), ((), ()))


def _tc_a_body(x_ref, WaF_ref, Wb_ref, A_ref, WbF_ref):
    A_ref[...] = lax.dot_general(x_ref[...], WaF_ref[...], DR,
                                 preferred_element_type=jnp.float32,
                                 precision=jax.lax.Precision.DEFAULT)

    @pl.when(pl.program_id(0) == 0)
    def _wbf():
        wt = jnp.transpose(Wb_ref[...], (0, 2, 1))
        WbF_ref[...] = wt.reshape(ER, OUT_FEATURES)


def _tc_a(flat_x, WaF, Wb, interpret=False):
    return pl.pallas_call(
        _tc_a_body,
        grid=(T // TB,),
        in_specs=[
            pl.BlockSpec((TB, IN_FEATURES), lambda i: (i, 0)),
            pl.BlockSpec((ER, IN_FEATURES), lambda i: (0, 0)),
            pl.BlockSpec((E, OUT_FEATURES, R), lambda i: (0, 0, 0)),
        ],
        out_specs=[
            pl.BlockSpec((TB, ER), lambda i: (i, 0)),
            pl.BlockSpec((ER, OUT_FEATURES), lambda i: (0, 0)),
        ],
        out_shape=[
            jax.ShapeDtypeStruct((T, ER), jnp.float32),
            jax.ShapeDtypeStruct((ER, OUT_FEATURES), jnp.float32),
        ],
        interpret=interpret,
    )(flat_x, WaF, Wb)


def _tc_body(A_ref, wa_ref, wb_ref, WbF_ref, pA_ref, pB_ref,
             out_ref, auxA_ref, auxB_ref):
    i = pl.program_id(0)
    f32 = jnp.float32
    hi = jax.lax.Precision.HIGHEST

    rows = lax.broadcasted_iota(jnp.int32, (E, ER), 0)
    cols = lax.broadcasted_iota(jnp.int32, (E, ER), 1)
    E1 = jnp.where(cols // R == rows, 1.0, 0.0).astype(f32)
    c1 = lax.broadcasted_iota(jnp.int32, (ER, ER), 0)
    c2 = lax.broadcasted_iota(jnp.int32, (ER, ER), 1)
    Km = jnp.where(c1 % R == c2 % R, 1.0, 0.0).astype(f32)

    lo = jax.lax.Precision.DEFAULT
    wa_exp = lax.dot_general(wa_ref[...], E1, DT,
                             preferred_element_type=f32, precision=hi)
    mid_tile = lax.dot_general(A_ref[...] * wa_exp, Km, DN,
                               preferred_element_type=f32, precision=hi)
    wb_exp = lax.dot_general(wb_ref[...], E1, DT,
                             preferred_element_type=f32, precision=hi)
    out_ref[...] = lax.dot_general(mid_tile * wb_exp, WbF_ref[...], DN,
                                   preferred_element_type=f32,
                                   precision=lo) * SCALING

    @pl.when(i == pl.num_programs(0) - 1)
    def _aux():
        for p_ref, aux_ref in ((pA_ref, auxA_ref), (pB_ref, auxB_ref)):
            p3 = p_ref[...].reshape(NW, E, 16)
            pm = jnp.sum(jnp.sum(p3, axis=2), axis=0, keepdims=True) / T
            mu = jnp.sum(pm) / E
            d = pm - mu
            aux_ref[...] = jnp.broadcast_to(E * jnp.sum(d * d) / (E - 1), (1, 1))


def _tc_combine(A, wa, wb, WbF, pA, pB, interpret=False):
    grid = (T // TB,)
    return pl.pallas_call(
        _tc_body,
        grid=grid,
        in_specs=[
            pl.BlockSpec((TB, ER), lambda i: (i, 0)),
            pl.BlockSpec((E, TB), lambda i: (0, i)),
            pl.BlockSpec((E, TB), lambda i: (0, i)),
            pl.BlockSpec((ER, OUT_FEATURES), lambda i: (0, 0)),
            pl.BlockSpec((NW * E, 16), lambda i: (0, 0)),
            pl.BlockSpec((NW * E, 16), lambda i: (0, 0)),
        ],
        out_specs=[
            pl.BlockSpec((TB, OUT_FEATURES), lambda i: (i, 0)),
            pl.BlockSpec((1, 1), lambda i: (0, 0)),
            pl.BlockSpec((1, 1), lambda i: (0, 0)),
        ],
        out_shape=[
            jax.ShapeDtypeStruct((T, OUT_FEATURES), jnp.float32),
            jax.ShapeDtypeStruct((1, 1), jnp.float32),
            jax.ShapeDtypeStruct((1, 1), jnp.float32),
        ],
        interpret=interpret,
    )(A, wa, wb, WbF, pA, pB)


def kernel(x, router_logits_a, router_logits_b, Wa, Wb):
    batch, seq, _ = x.shape
    flat_x = x.reshape(T, IN_FEATURES)
    laT = router_logits_a.T
    lbT = router_logits_b.T
    WaF = Wa.reshape(ER, IN_FEATURES)

    waT, wbT, pA, pB = _sc_route(laT, lbT)
    A, WbF = _tc_a(flat_x, WaF, Wb)
    out_flat, auxA, auxB = _tc_combine(A, waT, wbT, WbF, pA, pB)
    return (out_flat.reshape(batch, seq, OUT_FEATURES), auxA[0, 0], auxB[0, 0])


# speedup vs baseline: 1.2520x; 1.2301x over previous
import functools
import jax
import jax.numpy as jnp
from jax import lax
from jax.experimental import pallas as pl
from jax.experimental.pallas import tpu as pltpu
from jax.experimental.pallas import tpu_sc as plsc

IN_FEATURES = 2048
OUT_FEATURES = 2048
R = 8
LORA_ALPHA = 16
E = 8
TOP_K = 2
T = 4096
ER = E * R
SCALING = LORA_ALPHA / R

NW = 32
TPW = T // NW
NG = TPW // 16


def _route_group(l, acc):
    m = l[0]
    for e in range(1, E):
        m = jnp.maximum(m, l[e])
    ex = [jnp.exp(l[e] - m) for e in range(E)]
    total = ex[0]
    for e in range(1, E):
        total = total + ex[e]
    p = [ex[e] / total for e in range(E)]
    new_acc = [acc[e] + p[e] for e in range(E)]

    one = jnp.ones((16,), jnp.float32)
    zero = jnp.zeros((16,), jnp.float32)
    w = []
    sels = []
    for e in range(E):
        rank = zero
        for j in range(E):
            if j == e:
                continue
            gt = jnp.where(l[j] > l[e], one, zero)
            rank = rank + gt
            if j < e:
                rank = rank + jnp.where(l[j] == l[e], one, zero)
        sels.append(rank < (TOP_K - 0.5))
    denom = zero
    for e in range(E):
        denom = denom + jnp.where(sels[e], p[e], zero)
    for e in range(E):
        w.append(jnp.where(sels[e], p[e] / denom, zero))
    return w, new_acc


def _route_one(l_hbm, w_hbm, p_hbm, l_v, w_v, s_v, base, wid):
    pltpu.sync_copy(l_hbm.at[:, pl.ds(base, TPW)], l_v)

    def body(g, acc):
        l = [l_v[e, pl.ds(g * 16, 16)] for e in range(E)]
        w, acc = _route_group(l, list(acc))
        for e in range(E):
            w_v[e, pl.ds(g * 16, 16)] = w[e]
        return tuple(acc)

    zero = jnp.zeros((16,), jnp.float32)
    acc = lax.fori_loop(0, NG, body, tuple(zero for _ in range(E)))

    for e in range(E):
        s_v[e, :] = acc[e]

    pltpu.sync_copy(w_v, w_hbm.at[:, pl.ds(base, TPW)])
    pltpu.sync_copy(s_v, p_hbm.at[pl.ds(wid * E, E)])


def _sc_route_body(la, lb, wa, wb, pA, pB, la_v, wa_v, lb_v, wb_v, sa_v, sb_v):
    wid = lax.axis_index("s") * 2 + lax.axis_index("c")
    base = wid * TPW
    _route_one(la, wa, pA, la_v, wa_v, sa_v, base, wid)
    _route_one(lb, wb, pB, lb_v, wb_v, sb_v, base, wid)


_SC_ROUTE_CACHE = []


def _sc_route(laT, lbT):
    if not _SC_ROUTE_CACHE:
        _SC_ROUTE_CACHE.append(functools.partial(
            pl.kernel,
            mesh=plsc.VectorSubcoreMesh(core_axis_name="c", subcore_axis_name="s"),
            out_type=[
                jax.ShapeDtypeStruct((E, T), jnp.float32),
                jax.ShapeDtypeStruct((E, T), jnp.float32),
                jax.ShapeDtypeStruct((NW * E, 16), jnp.float32),
                jax.ShapeDtypeStruct((NW * E, 16), jnp.float32),
            ],
            scratch_types=[
                pltpu.VMEM((E, TPW), jnp.float32),
                pltpu.VMEM((E, TPW), jnp.float32),
                pltpu.VMEM((E, TPW), jnp.float32),
                pltpu.VMEM((E, TPW), jnp.float32),
                pltpu.VMEM((E, 16), jnp.float32),
                pltpu.VMEM((E, 16), jnp.float32),
            ],
        )(_sc_route_body))
    return _SC_ROUTE_CACHE[0](laT, lbT)


TB = 1024
DN = (((1,), (0,)), ((), ()))
DT = (((0,), (0,)), ((), ()))


DR = (((1,), (1,)), ((), ()))


def _tc_a_body(x_ref, WaF_ref, A_ref):
    A_ref[...] = lax.dot_general(x_ref[...], WaF_ref[...], DR,
                                 preferred_element_type=jnp.float32,
                                 precision=jax.lax.Precision.DEFAULT)


def _tc_a(flat_x, WaF, interpret=False):
    return pl.pallas_call(
        _tc_a_body,
        grid=(T // TB,),
        in_specs=[
            pl.BlockSpec((TB, IN_FEATURES), lambda i: (i, 0)),
            pl.BlockSpec((ER, IN_FEATURES), lambda i: (0, 0)),
        ],
        out_specs=pl.BlockSpec((TB, ER), lambda i: (i, 0)),
        out_shape=jax.ShapeDtypeStruct((T, ER), jnp.float32),
        interpret=interpret,
    )(flat_x, WaF)


def _tc_body(A_ref, wa_ref, wb_ref, WbF_ref, pA_ref, pB_ref,
             out_ref, auxA_ref, auxB_ref):
    i = pl.program_id(0)
    f32 = jnp.float32
    hi = jax.lax.Precision.HIGHEST

    rows = lax.broadcasted_iota(jnp.int32, (E, ER), 0)
    cols = lax.broadcasted_iota(jnp.int32, (E, ER), 1)
    E1 = jnp.where(cols // R == rows, 1.0, 0.0).astype(f32)
    c1 = lax.broadcasted_iota(jnp.int32, (ER, ER), 0)
    c2 = lax.broadcasted_iota(jnp.int32, (ER, ER), 1)
    Km = jnp.where(c1 % R == c2 % R, 1.0, 0.0).astype(f32)

    lo = jax.lax.Precision.DEFAULT
    wa_exp = lax.dot_general(wa_ref[...], E1, DT,
                             preferred_element_type=f32, precision=hi)
    mid_tile = lax.dot_general(A_ref[...] * wa_exp, Km, DN,
                               preferred_element_type=f32, precision=hi)
    wb_exp = lax.dot_general(wb_ref[...], E1, DT,
                             preferred_element_type=f32, precision=hi)
    out_ref[...] = lax.dot_general(mid_tile * wb_exp, WbF_ref[...], DN,
                                   preferred_element_type=f32,
                                   precision=lo) * SCALING

    @pl.when(i == pl.num_programs(0) - 1)
    def _aux():
        for p_ref, aux_ref in ((pA_ref, auxA_ref), (pB_ref, auxB_ref)):
            p3 = p_ref[...].reshape(NW, E, 16)
            pm = jnp.sum(jnp.sum(p3, axis=2), axis=0, keepdims=True) / T
            mu = jnp.sum(pm) / E
            d = pm - mu
            aux_ref[...] = jnp.broadcast_to(E * jnp.sum(d * d) / (E - 1), (1, 1))


def _tc_combine(A, wa, wb, WbF, pA, pB, interpret=False):
    grid = (T // TB,)
    return pl.pallas_call(
        _tc_body,
        grid=grid,
        in_specs=[
            pl.BlockSpec((TB, ER), lambda i: (i, 0)),
            pl.BlockSpec((E, TB), lambda i: (0, i)),
            pl.BlockSpec((E, TB), lambda i: (0, i)),
            pl.BlockSpec((ER, OUT_FEATURES), lambda i: (0, 0)),
            pl.BlockSpec((NW * E, 16), lambda i: (0, 0)),
            pl.BlockSpec((NW * E, 16), lambda i: (0, 0)),
        ],
        out_specs=[
            pl.BlockSpec((TB, OUT_FEATURES), lambda i: (i, 0)),
            pl.BlockSpec((1, 1), lambda i: (0, 0)),
            pl.BlockSpec((1, 1), lambda i: (0, 0)),
        ],
        out_shape=[
            jax.ShapeDtypeStruct((T, OUT_FEATURES), jnp.float32),
            jax.ShapeDtypeStruct((1, 1), jnp.float32),
            jax.ShapeDtypeStruct((1, 1), jnp.float32),
        ],
        interpret=interpret,
    )(A, wa, wb, WbF, pA, pB)


def kernel(x, router_logits_a, router_logits_b, Wa, Wb):
    batch, seq, _ = x.shape
    flat_x = x.reshape(T, IN_FEATURES)
    laT = router_logits_a.T
    lbT = router_logits_b.T
    WaF = Wa.reshape(ER, IN_FEATURES)
    WbF = Wb.transpose(0, 2, 1).reshape(ER, OUT_FEATURES)

    waT, wbT, pA, pB = _sc_route(laT, lbT)
    A = _tc_a(flat_x, WaF)
    out_flat, auxA, auxB = _tc_combine(A, waT, wbT, WbF, pA, pB)
    return (out_flat.reshape(batch, seq, OUT_FEATURES), auxA[0, 0], auxB[0, 0])


# speedup vs baseline: 1.2522x; 1.0001x over previous
import functools
import jax
import jax.numpy as jnp
from jax import lax
from jax.experimental import pallas as pl
from jax.experimental.pallas import tpu as pltpu
from jax.experimental.pallas import tpu_sc as plsc

IN_FEATURES = 2048
OUT_FEATURES = 2048
R = 8
LORA_ALPHA = 16
E = 8
TOP_K = 2
T = 4096
ER = E * R
SCALING = LORA_ALPHA / R

NW = 32
TPW = T // NW
NG = TPW // 16


def _route_group(l, acc):
    m = l[0]
    for e in range(1, E):
        m = jnp.maximum(m, l[e])
    ex = [jnp.exp(l[e] - m) for e in range(E)]
    total = ex[0]
    for e in range(1, E):
        total = total + ex[e]
    p = [ex[e] / total for e in range(E)]
    new_acc = [acc[e] + p[e] for e in range(E)]

    one = jnp.ones((16,), jnp.float32)
    zero = jnp.zeros((16,), jnp.float32)
    w = []
    sels = []
    for e in range(E):
        rank = zero
        for j in range(E):
            if j == e:
                continue
            gt = jnp.where(l[j] > l[e], one, zero)
            rank = rank + gt
            if j < e:
                rank = rank + jnp.where(l[j] == l[e], one, zero)
        sels.append(rank < (TOP_K - 0.5))
    denom = zero
    for e in range(E):
        denom = denom + jnp.where(sels[e], p[e], zero)
    for e in range(E):
        w.append(jnp.where(sels[e], p[e] / denom, zero))
    return w, new_acc


def _route_one(l_hbm, w_hbm, p_hbm, l_v, w_v, s_v, base, wid):
    pltpu.sync_copy(l_hbm.at[:, pl.ds(base, TPW)], l_v)

    def body(g, acc):
        l = [l_v[e, pl.ds(g * 16, 16)] for e in range(E)]
        w, acc = _route_group(l, list(acc))
        for e in range(E):
            w_v[e, pl.ds(g * 16, 16)] = w[e]
        return tuple(acc)

    zero = jnp.zeros((16,), jnp.float32)
    acc = lax.fori_loop(0, NG, body, tuple(zero for _ in range(E)))

    for e in range(E):
        s_v[e, :] = acc[e]

    pltpu.sync_copy(w_v, w_hbm.at[:, pl.ds(base, TPW)])
    pltpu.sync_copy(s_v, p_hbm.at[pl.ds(wid * E, E)])


def _sc_route_body(la, lb, wa, wb, pA, pB, la_v, wa_v, lb_v, wb_v, sa_v, sb_v):
    wid = lax.axis_index("s") * 2 + lax.axis_index("c")
    base = wid * TPW
    _route_one(la, wa, pA, la_v, wa_v, sa_v, base, wid)
    _route_one(lb, wb, pB, lb_v, wb_v, sb_v, base, wid)


_SC_ROUTE_CACHE = []


def _sc_route(laT, lbT):
    if not _SC_ROUTE_CACHE:
        _SC_ROUTE_CACHE.append(functools.partial(
            pl.kernel,
            mesh=plsc.VectorSubcoreMesh(core_axis_name="c", subcore_axis_name="s"),
            out_type=[
                jax.ShapeDtypeStruct((E, T), jnp.float32),
                jax.ShapeDtypeStruct((E, T), jnp.float32),
                jax.ShapeDtypeStruct((NW * E, 16), jnp.float32),
                jax.ShapeDtypeStruct((NW * E, 16), jnp.float32),
            ],
            scratch_types=[
                pltpu.VMEM((E, TPW), jnp.float32),
                pltpu.VMEM((E, TPW), jnp.float32),
                pltpu.VMEM((E, TPW), jnp.float32),
                pltpu.VMEM((E, TPW), jnp.float32),
                pltpu.VMEM((E, 16), jnp.float32),
                pltpu.VMEM((E, 16), jnp.float32),
            ],
        )(_sc_route_body))
    return _SC_ROUTE_CACHE[0](laT, lbT)


TB = 1024
DN = (((1,), (0,)), ((), ()))
DT = (((0,), (0,)), ((), ()))


DR = (((1,), (1,)), ((), ()))


def _tc_a_body(x_ref, WaF_ref, A_ref):
    A_ref[...] = lax.dot_general(x_ref[...], WaF_ref[...], DR,
                                 preferred_element_type=jnp.float32,
                                 precision=jax.lax.Precision.DEFAULT)


def _tc_a(flat_x, WaF):
    return pl.pallas_call(
        _tc_a_body,
        grid=(T // TB,),
        in_specs=[
            pl.BlockSpec((TB, IN_FEATURES), lambda i: (i, 0)),
            pl.BlockSpec((ER, IN_FEATURES), lambda i: (0, 0)),
        ],
        out_specs=pl.BlockSpec((TB, ER), lambda i: (i, 0)),
        out_shape=jax.ShapeDtypeStruct((T, ER), jnp.float32),
    )(flat_x, WaF)


def _tc_body(A_ref, wa_ref, wb_ref, WbF_ref, pA_ref, pB_ref,
             out_ref, auxA_ref, auxB_ref):
    i = pl.program_id(0)
    f32 = jnp.float32
    hi = jax.lax.Precision.HIGHEST

    rows = lax.broadcasted_iota(jnp.int32, (E, ER), 0)
    cols = lax.broadcasted_iota(jnp.int32, (E, ER), 1)
    E1 = jnp.where(cols // R == rows, 1.0, 0.0).astype(f32)
    c1 = lax.broadcasted_iota(jnp.int32, (ER, ER), 0)
    c2 = lax.broadcasted_iota(jnp.int32, (ER, ER), 1)
    Km = jnp.where(c1 % R == c2 % R, 1.0, 0.0).astype(f32)

    lo = jax.lax.Precision.DEFAULT
    wa_exp = lax.dot_general(wa_ref[...], E1, DT,
                             preferred_element_type=f32, precision=hi)
    mid_tile = lax.dot_general(A_ref[...] * wa_exp, Km, DN,
                               preferred_element_type=f32, precision=hi)
    wb_exp = lax.dot_general(wb_ref[...], E1, DT,
                             preferred_element_type=f32, precision=hi)
    out_ref[...] = lax.dot_general(mid_tile * wb_exp, WbF_ref[...], DN,
                                   preferred_element_type=f32,
                                   precision=lo) * SCALING

    @pl.when(i == pl.num_programs(0) - 1)
    def _aux():
        for p_ref, aux_ref in ((pA_ref, auxA_ref), (pB_ref, auxB_ref)):
            p3 = p_ref[...].reshape(NW, E, 16)
            pm = jnp.sum(jnp.sum(p3, axis=2), axis=0, keepdims=True) / T
            mu = jnp.sum(pm) / E
            d = pm - mu
            aux_ref[...] = jnp.broadcast_to(E * jnp.sum(d * d) / (E - 1), (1, 1))


def _tc_combine(A, wa, wb, WbF, pA, pB):
    grid = (T // TB,)
    return pl.pallas_call(
        _tc_body,
        grid=grid,
        in_specs=[
            pl.BlockSpec((TB, ER), lambda i: (i, 0)),
            pl.BlockSpec((E, TB), lambda i: (0, i)),
            pl.BlockSpec((E, TB), lambda i: (0, i)),
            pl.BlockSpec((ER, OUT_FEATURES), lambda i: (0, 0)),
            pl.BlockSpec((NW * E, 16), lambda i: (0, 0)),
            pl.BlockSpec((NW * E, 16), lambda i: (0, 0)),
        ],
        out_specs=[
            pl.BlockSpec((TB, OUT_FEATURES), lambda i: (i, 0)),
            pl.BlockSpec((1, 1), lambda i: (0, 0)),
            pl.BlockSpec((1, 1), lambda i: (0, 0)),
        ],
        out_shape=[
            jax.ShapeDtypeStruct((T, OUT_FEATURES), jnp.float32),
            jax.ShapeDtypeStruct((1, 1), jnp.float32),
            jax.ShapeDtypeStruct((1, 1), jnp.float32),
        ],
    )(A, wa, wb, WbF, pA, pB)


def kernel(x, router_logits_a, router_logits_b, Wa, Wb):
    batch, seq, _ = x.shape
    flat_x = x.reshape(T, IN_FEATURES)
    laT = router_logits_a.T
    lbT = router_logits_b.T
    WaF = Wa.reshape(ER, IN_FEATURES)
    WbF = Wb.transpose(0, 2, 1).reshape(ER, OUT_FEATURES)

    waT, wbT, pA, pB = _sc_route(laT, lbT)
    A = _tc_a(flat_x, WaF)
    out_flat, auxA, auxB = _tc_combine(A, waT, wbT, WbF, pA, pB)
    return (out_flat.reshape(batch, seq, OUT_FEATURES), auxA[0, 0], auxB[0, 0])
